# Initial kernel scaffold; baseline (speedup 1.0000x reference)
import jax
import jax.numpy as jnp
from jax.experimental import pallas as pl


def kernel(pred, target):
    raise NotImplementedError("write your pallas kernel here")


# speedup vs baseline: 24.4567x; 24.4567x over previous
import functools

import jax
import jax.numpy as jnp
from jax import lax
from jax.experimental import pallas as pl
from jax.experimental.pallas import tpu as pltpu
from jax.experimental.pallas import tpu_sc as plsc

_NIMG = 4
_NCLS = 8
_NPIX = 224 * 224
_NIC = _NIMG * _NCLS
_B = 8192
_CHUNK = _NPIX // 16


def _softmax_body(x_ref, o_ref):
    x = x_ref[0]
    m = jnp.max(x, axis=0, keepdims=True)
    e = jnp.exp(x - m)
    s = jnp.sum(e, axis=0, keepdims=True)
    o_ref[0] = e / s


def _softmax(pred3):
    c = 6272
    return pl.pallas_call(
        _softmax_body,
        grid=(_NIMG, _NPIX // c),
        in_specs=[pl.BlockSpec((1, _NCLS, c), lambda i, j: (i, 0, j))],
        out_specs=pl.BlockSpec((1, _NCLS, c), lambda i, j: (i, 0, j)),
        out_shape=jax.ShapeDtypeStruct((_NIMG, _NCLS, _NPIX), jnp.float32),
    )(pred3)


_sc_mesh = plsc.VectorSubcoreMesh(core_axis_name="c", subcore_axis_name="s")


@functools.partial(
    pl.kernel,
    out_type=(
        jax.ShapeDtypeStruct((_NIC, _B), jnp.float32),
        jax.ShapeDtypeStruct((_NIC, 16), jnp.float32),
    ),
    mesh=_sc_mesh,
    compiler_params=pltpu.CompilerParams(needs_layout_passes=False),
    scratch_types=[
        pltpu.VMEM((_NPIX,), jnp.float32),
        pltpu.VMEM((_NPIX,), jnp.int32),
        pltpu.VMEM((_B,), jnp.float32),
        pltpu.VMEM((16,), jnp.float32),
    ],
)
def _sc_hist(p_hbm, t_hbm, cnt_hbm, aux_hbm, p_v, t_v, cnt_v, aux_v):
    wid = lax.axis_index("s") * 2 + lax.axis_index("c")
    img = wid // _NCLS
    cls = wid % _NCLS

    zeros16 = jnp.zeros((16,), jnp.float32)
    ones16 = jnp.ones((16,), jnp.float32)

    def zero_body(k, carry):
        cnt_v[pl.ds(pl.multiple_of(k * 16, 16), 16)] = zeros16
        return carry

    lax.fori_loop(0, _B // 16, zero_body, 0)

    pltpu.sync_copy(p_hbm.at[wid], p_v)
    pltpu.sync_copy(t_hbm.at[img], t_v)

    bf = jnp.float32(_B)

    def body(j, sum_pos):
        off = pl.multiple_of(j * 16, 16)
        p16 = p_v[pl.ds(off, 16)]
        t16 = t_v[pl.ds(off, 16)]
        is_pos = t16 == cls
        b16 = jnp.minimum((p16 * bf).astype(jnp.int32), _B - 1)
        plsc.addupdate_scatter(cnt_v, [b16], ones16,
                               mask=jnp.logical_not(is_pos))
        return sum_pos + jnp.where(is_pos, p16, 0.0)

    sum_pos = lax.fori_loop(0, _CHUNK, body, zeros16)
    sp = jnp.sum(sum_pos, axis=0)
    lane = lax.iota(jnp.int32, 16)
    aux_v[...] = jnp.where(lane == 0, sp, 0.0)

    pltpu.sync_copy(cnt_v, cnt_hbm.at[wid])
    pltpu.sync_copy(aux_v, aux_hbm.at[wid])


def _finalize_body(cnt_ref, aux_ref, o_ref):
    cnt = cnt_ref[...]
    npixf = jnp.float32(_NPIX)
    n_neg = jnp.sum(cnt, axis=1, keepdims=True)
    p_cnt = npixf - n_neg

    csum = cnt
    d = 1
    while d < _B:
        shifted = jnp.concatenate(
            [jnp.zeros((_NIC, d), jnp.float32), csum[:, : _B - d]], axis=1)
        csum = csum + shifted
        d *= 2

    k_above = n_neg - csum
    pk = p_cnt + k_above
    d_w = p_cnt * cnt / (jnp.maximum(pk, 1.0) * (pk + cnt))
    d_w = d_w + jnp.where((p_cnt == 0.0) & (k_above == 0.0) & (cnt > 0.0),
                          1.0, 0.0)
    mid = (lax.broadcasted_iota(jnp.int32, (_NIC, _B), 1).astype(jnp.float32)
           + 0.5) / _B
    neg_part = jnp.sum(d_w * (1.0 + mid), axis=1, keepdims=True)

    sum_pos = aux_ref[...][:, 0:1]
    loss = neg_part + (p_cnt - sum_pos) / npixf
    o_ref[...] = jnp.sum(loss, axis=(0, 1), keepdims=True) / jnp.float32(_NIC)


def _finalize(cnt, aux):
    return pl.pallas_call(
        _finalize_body,
        out_shape=jax.ShapeDtypeStruct((1, 1), jnp.float32),
    )(cnt, aux)


def kernel(pred, target):
    probs = _softmax(pred.reshape(_NIMG, _NCLS, _NPIX))
    cnt, aux = _sc_hist(
        probs.reshape(_NIC, _NPIX),
        target.reshape(_NIMG, _NPIX).astype(jnp.int32),
    )
    return _finalize(cnt, aux)[0, 0]


# speedup vs baseline: 27.2692x; 1.1150x over previous
import functools

import jax
import jax.numpy as jnp
from jax import lax
from jax.experimental import pallas as pl
from jax.experimental.pallas import tpu as pltpu
from jax.experimental.pallas import tpu_sc as plsc

_NIMG = 4
_NCLS = 8
_NPIX = 224 * 224
_NIC = _NIMG * _NCLS
_B = 8192
_BPAD = _B + 16
_CLAMP = (_B - 0.5) / _B


def _prep_body(x_ref, t_ref, v_ref, pos_ref):
    x = x_ref[0]
    t = t_ref[0]
    m = jnp.max(x, axis=0, keepdims=True)
    e = jnp.exp(x - m)
    p = e / jnp.sum(e, axis=0, keepdims=True)
    cls = lax.broadcasted_iota(jnp.int32, (_NCLS, _NPIX), 0)
    is_pos = t == cls
    v_ref[0] = jnp.where(is_pos, jnp.float32(2.0),
                         jnp.minimum(p, jnp.float32(_CLAMP)))
    ps = jnp.sum(jnp.where(is_pos, p, 0.0), axis=1, keepdims=True)
    lane = lax.broadcasted_iota(jnp.int32, (_NCLS, 128), 1)
    pos_ref[0] = jnp.where(lane == 0, ps, 0.0)


def _prep(pred3, target3):
    return pl.pallas_call(
        _prep_body,
        grid=(_NIMG,),
        in_specs=[
            pl.BlockSpec((1, _NCLS, _NPIX), lambda i: (i, 0, 0)),
            pl.BlockSpec((1, 1, _NPIX), lambda i: (i, 0, 0)),
        ],
        out_specs=[
            pl.BlockSpec((1, _NCLS, _NPIX), lambda i: (i, 0, 0)),
            pl.BlockSpec((1, _NCLS, 128), lambda i: (i, 0, 0)),
        ],
        out_shape=[
            jax.ShapeDtypeStruct((_NIMG, _NCLS, _NPIX), jnp.float32),
            jax.ShapeDtypeStruct((_NIMG, _NCLS, 128), jnp.float32),
        ],
    )(pred3, target3)


_sc_mesh = plsc.VectorSubcoreMesh(core_axis_name="c", subcore_axis_name="s")


@functools.partial(
    pl.kernel,
    out_type=jax.ShapeDtypeStruct((_NIC, _B), jnp.float32),
    mesh=_sc_mesh,
    compiler_params=pltpu.CompilerParams(needs_layout_passes=False),
    scratch_types=[
        pltpu.VMEM((_NPIX,), jnp.float32),
        pltpu.VMEM((_BPAD,), jnp.float32),
    ],
)
def _sc_hist(v_hbm, cnt_hbm, v_v, cnt_v):
    wid = lax.axis_index("s") * 2 + lax.axis_index("c")

    zeros16 = jnp.zeros((16,), jnp.float32)
    ones16 = jnp.ones((16,), jnp.float32)
    bf = jnp.float32(_B)

    def _zero(k, carry):
        cnt_v[pl.ds(pl.multiple_of(k * 16, 16), 16)] = zeros16
        return carry

    lax.fori_loop(0, _BPAD // 16, _zero, 0)

    pltpu.sync_copy(v_hbm.at[wid], v_v)

    def _hist(j, carry):
        off = pl.multiple_of(j * 16, 16)
        v16 = v_v[pl.ds(off, 16)]
        b16 = jnp.minimum((v16 * bf).astype(jnp.int32), _B)
        plsc.addupdate_scatter(cnt_v, [b16], ones16)
        return carry

    lax.fori_loop(0, _NPIX // 16, _hist, 0)

    pltpu.sync_copy(cnt_v.at[pl.ds(0, _B)], cnt_hbm.at[wid])


def _finalize_body(cnt_ref, pos_ref, o_ref):
    cnt = cnt_ref[...]
    npixf = jnp.float32(_NPIX)
    n_neg = jnp.sum(cnt, axis=1, keepdims=True)
    p_cnt = npixf - n_neg

    csum = cnt
    d = 1
    while d < _B:
        shifted = jnp.concatenate(
            [jnp.zeros((_NIC, d), jnp.float32), csum[:, : _B - d]], axis=1)
        csum = csum + shifted
        d *= 2

    k_above = n_neg - csum
    pk = p_cnt + k_above
    d_w = p_cnt * cnt / (jnp.maximum(pk, 1.0) * (pk + cnt))
    d_w = d_w + jnp.where((p_cnt == 0.0) & (k_above == 0.0) & (cnt > 0.0),
                          1.0, 0.0)
    mid = (lax.broadcasted_iota(jnp.int32, (_NIC, _B), 1).astype(jnp.float32)
           + 0.5) / _B
    neg_part = jnp.sum(d_w * (1.0 + mid), axis=1, keepdims=True)

    sum_pos = pos_ref[...][:, 0:1]
    loss = neg_part + (p_cnt - sum_pos) / npixf
    o_ref[...] = jnp.sum(loss, axis=(0, 1), keepdims=True) / jnp.float32(_NIC)


def _finalize(cnt, pos):
    return pl.pallas_call(
        _finalize_body,
        out_shape=jax.ShapeDtypeStruct((1, 1), jnp.float32),
    )(cnt, pos)


def kernel(pred, target):
    pred3 = pred.reshape(_NIMG, _NCLS, _NPIX)
    target3 = target.reshape(_NIMG, 1, _NPIX).astype(jnp.int32)
    v, pos = _prep(pred3, target3)
    cnt = _sc_hist(v.reshape(_NIC, _NPIX))
    return _finalize(cnt, pos.reshape(_NIC, 128))[0, 0]
